# Initial kernel scaffold; baseline (speedup 1.0000x reference)
import jax
import jax.numpy as jnp
from jax.experimental import pallas as pl


def kernel(x, edge_index, assume_mat, pooled_edge_index, batch, W_pre1, b_pre1, g1, be1, W_pre2, b_pre2, g2, be2, W_g1, b_g1, g3, be3, W_g2, b_g2, g4, be4, W_lin, b_lin):
    raise NotImplementedError("write your pallas kernel here")


# speedup vs baseline: 1.0801x; 1.0801x over previous
import jax
import jax.numpy as jnp
from jax import lax
from jax.experimental import pallas as pl
from jax.experimental.pallas import tpu as pltpu

_EDGE_CHUNK = 2000


def _deg_body(dst_ref, out_ref):
    ec = pl.program_id(0)
    nc = pl.num_programs(0)

    @pl.when(ec == 0)
    def _():
        out_ref[...] = jnp.zeros_like(out_ref)

    def body(i, carry):
        d = dst_ref[0, 0, i]
        out_ref[pl.ds(d, 1), :] += 1.0
        return carry

    lax.fori_loop(0, dst_ref.shape[2], body, 0)

    @pl.when(ec == nc - 1)
    def _():
        out_ref[...] = lax.rsqrt(out_ref[...] + 1.0)


def _deg_call(dst, n):
    e = dst.shape[0]
    c = _EDGE_CHUNK if e % _EDGE_CHUNK == 0 else e
    nch = e // c
    dst3 = dst.reshape(nch, 1, c)
    return pl.pallas_call(
        _deg_body,
        grid=(nch,),
        in_specs=[pl.BlockSpec((1, 1, c), lambda ec: (ec, 0, 0), memory_space=pltpu.SMEM)],
        out_specs=pl.BlockSpec((n, 1), lambda ec: (0, 0)),
        out_shape=jax.ShapeDtypeStruct((n, 1), jnp.float32),
    )(dst3)


def _mm_scale_body(x_ref, w_ref, s_ref, out_ref):
    out_ref[...] = (
        jnp.dot(x_ref[...], w_ref[...], preferred_element_type=jnp.float32) * s_ref[...]
    )


def _mm_scale_call(x, w, scale, rb):
    n, k = x.shape
    h = w.shape[1]
    nb = n // rb
    return pl.pallas_call(
        _mm_scale_body,
        grid=(nb,),
        in_specs=[
            pl.BlockSpec((rb, k), lambda i: (i, 0)),
            pl.BlockSpec((k, h), lambda i: (0, 0)),
            pl.BlockSpec((rb, 1), lambda i: (i, 0)),
        ],
        out_specs=pl.BlockSpec((rb, h), lambda i: (i, 0)),
        out_shape=jax.ShapeDtypeStruct((n, h), jnp.float32),
    )(x, w, scale)


def _agg_body(hs_ref, src_ref, dst_ref, dinv_ref, b_ref, g_ref, be_ref, out_ref):
    ec = pl.program_id(0)
    nc = pl.num_programs(0)

    @pl.when(ec == 0)
    def _():
        out_ref[...] = jnp.zeros_like(out_ref)

    def body(i, carry):
        s = src_ref[0, 0, i]
        d = dst_ref[0, 0, i]
        out_ref[pl.ds(d, 1), :] += hs_ref[pl.ds(s, 1), :]
        return carry

    lax.fori_loop(0, src_ref.shape[2], body, 0)

    @pl.when(ec == nc - 1)
    def _():
        v = dinv_ref[...] * (out_ref[...] + hs_ref[...]) + b_ref[...]
        m = jnp.mean(v, axis=0, keepdims=True)
        var = jnp.mean((v - m) ** 2, axis=0, keepdims=True)
        vn = (v - m) * lax.rsqrt(var + 1e-5) * g_ref[...] + be_ref[...]
        out_ref[...] = jnp.maximum(vn, 0.0)


def _agg_call(hs, src, dst, dinv, b, g, be):
    n, h = hs.shape
    e = src.shape[0]
    c = _EDGE_CHUNK if e % _EDGE_CHUNK == 0 else e
    nch = e // c
    src3 = src.reshape(nch, 1, c)
    dst3 = dst.reshape(nch, 1, c)
    b2 = b.reshape(1, h)
    g2 = g.reshape(1, h)
    be2 = be.reshape(1, h)
    idx_spec = pl.BlockSpec((1, 1, c), lambda ec: (ec, 0, 0), memory_space=pltpu.SMEM)
    full = lambda shape: pl.BlockSpec(shape, lambda ec: tuple(0 for _ in shape))
    return pl.pallas_call(
        _agg_body,
        grid=(nch,),
        in_specs=[
            full((n, h)),
            idx_spec,
            idx_spec,
            full((n, 1)),
            full((1, h)),
            full((1, h)),
            full((1, h)),
        ],
        out_specs=full((n, h)),
        out_shape=jax.ShapeDtypeStruct((n, h), jnp.float32),
    )(hs, src3, dst3, dinv, b2, g2, be2)


def _poolmm_body(a_ref, h_ref, out_ref):
    out_ref[...] = jnp.dot(a_ref[...], h_ref[...], preferred_element_type=jnp.float32)


def _poolmm_call(a, h):
    np_, n = a.shape
    hh = h.shape[1]
    rb = 160
    pad = (-np_) % rb
    if pad:
        a = jnp.pad(a, ((0, pad), (0, 0)))
    rows = np_ + pad
    out = pl.pallas_call(
        _poolmm_body,
        grid=(rows // rb,),
        in_specs=[
            pl.BlockSpec((rb, n), lambda i: (i, 0)),
            pl.BlockSpec((n, hh), lambda i: (0, 0)),
        ],
        out_specs=pl.BlockSpec((rb, hh), lambda i: (i, 0)),
        out_shape=jax.ShapeDtypeStruct((rows, hh), jnp.float32),
    )(a, h)
    return out[:np_] if pad else out


def _seg_body(hp_ref, batch_ref, wl_ref, bl_ref, out_ref):
    hp = hp_ref[...]
    batch = batch_ref[...]
    ng = out_ref.shape[0]
    neg = jnp.finfo(jnp.float32).min
    mxs = []
    mns = []
    for gidx in range(ng):
        mask = batch == gidx
        mx = jnp.max(jnp.where(mask, hp, neg), axis=0, keepdims=True)
        cnt = jnp.sum(mask.astype(jnp.float32))
        sm = jnp.sum(jnp.where(mask, hp, 0.0), axis=0, keepdims=True)
        mxs.append(mx)
        mns.append(sm / jnp.maximum(cnt, 1.0))
    rep = jnp.concatenate(
        [jnp.concatenate(mxs, axis=0), jnp.concatenate(mns, axis=0)], axis=1
    )
    out_ref[...] = (
        jnp.dot(rep, wl_ref[...], preferred_element_type=jnp.float32) + bl_ref[...]
    )


def _seg_call(hp, batch, w_lin, b_lin, ng):
    np_, h = hp.shape
    nc = w_lin.shape[1]
    batch2 = batch.reshape(np_, 1)
    bl2 = b_lin.reshape(1, nc)
    full = lambda shape: pl.BlockSpec(shape, lambda: tuple(0 for _ in shape))
    return pl.pallas_call(
        _seg_body,
        in_specs=[
            full((np_, h)),
            full((np_, 1)),
            full((2 * h, nc)),
            full((1, nc)),
        ],
        out_specs=full((ng, nc)),
        out_shape=jax.ShapeDtypeStruct((ng, nc), jnp.float32),
    )(hp, batch2, w_lin, bl2)


def kernel(x, edge_index, assume_mat, pooled_edge_index, batch, W_pre1, b_pre1, g1, be1,
           W_pre2, b_pre2, g2, be2, W_g1, b_g1, g3, be3, W_g2, b_g2, g4, be4,
           W_lin, b_lin):
    n = x.shape[0]
    np_ = assume_mat.shape[0]
    ng = 16
    src, dst = edge_index[0], edge_index[1]
    ps, pd = pooled_edge_index[0], pooled_edge_index[1]

    rb = 2000 if n % 2000 == 0 else n

    dinv = _deg_call(dst, n)
    hs1 = _mm_scale_call(x, W_pre1, dinv, rb)
    h1 = _agg_call(hs1, src, dst, dinv, b_pre1, g1, be1)
    hs2 = _mm_scale_call(h1, W_pre2, dinv, rb)
    h2 = _agg_call(hs2, src, dst, dinv, b_pre2, g2, be2)

    hp0 = _poolmm_call(assume_mat, h2)

    dinvp = _deg_call(pd, np_)
    hps1 = _mm_scale_call(hp0, W_g1, dinvp, np_)
    hp1 = _agg_call(hps1, ps, pd, dinvp, b_g1, g3, be3)
    hps2 = _mm_scale_call(hp1, W_g2, dinvp, np_)
    hp2 = _agg_call(hps2, ps, pd, dinvp, b_g2, g4, be4)

    return _seg_call(hp2, batch, W_lin, b_lin, ng)
